# baseline (speedup 1.0000x reference)
import functools

import jax
import jax.numpy as jnp
from jax import lax
from jax.experimental import pallas as pl
from jax.experimental.pallas import tpu as pltpu
from jax.experimental.pallas import tpu_sc as plsc

NC = 2
NS = 16
LN = 16
EB = 128


def _ceil_to(x, m):
    return (x + m - 1) // m * m


def _make_w_kernel(n_pad, e_pad, heads):
    per_w = e_pad // (NC * NS)
    n_chunks = per_w // EB
    st_rows = _ceil_to(_ceil_to(n_pad * heads, 128) // 128, NS)
    s_chunks = []
    b = 0
    while b < st_rows:
        sz = min(128, st_rows - b)
        s_chunks.append((b, sz))
        b += sz

    mesh = plsc.VectorSubcoreMesh(core_axis_name="c", subcore_axis_name="s")

    def body(src_hbm, dst_hbm, asrc_hbm, adst_hbm, zeros_hbm,
             w_hbm, ps_hbm, s_sh, asrc_v, adst_v, s_v, srcv, dstv,
             wbufs, idxbufs):
        c = lax.axis_index("c")
        s = lax.axis_index("s")
        wid = c * NS + s
        lanes = lax.broadcasted_iota(jnp.int32, (LN,), 0)
        zf = jnp.zeros((LN,), jnp.float32)

        pltpu.sync_copy(asrc_hbm, asrc_v)
        pltpu.sync_copy(adst_hbm, adst_v)

        @pl.when(s == 0)
        def _():
            pltpu.sync_copy(zeros_hbm.at[pl.ds(0, st_rows)], s_sh)

        def z_body(r, carry):
            for j in range(128 // LN):
                s_v[r, pl.ds(j * LN, LN)] = zf
            return carry

        lax.fori_loop(0, st_rows, z_body, 0)
        for bi, (base, sz) in enumerate(s_chunks):
            for g in range(sz // LN):
                idxbufs[bi][pl.ds(g * LN, LN)] = lanes + (base + g * LN)

        plsc.subcore_barrier()

        def chunk_body(k, carry):
            ebase = wid * per_w + k * EB
            pltpu.sync_copy(src_hbm.at[pl.ds(ebase, EB)], srcv)
            pltpu.sync_copy(dst_hbm.at[pl.ds(ebase, EB)], dstv)

            for g in range(EB // LN):
                s16 = srcv[pl.ds(g * LN, LN)]
                d16 = dstv[pl.ds(g * LN, LN)]
                for hd in range(heads):
                    a = plsc.load_gather(asrc_v, [s16 * heads + hd])
                    bb = plsc.load_gather(adst_v, [d16 * heads + hd])
                    al = a + bb
                    al = jnp.where(al >= 0.0, al, al * 0.2)
                    wbufs[hd][pl.ds(g * LN, LN)] = jnp.exp(al)
            for hd in range(heads):
                pltpu.sync_copy(wbufs[hd], w_hbm.at[hd, pl.ds(ebase, EB)])

            def e_body(e, carry2):
                zi = jnp.zeros((LN,), jnp.int32) + e
                wcomb = zf
                for hd in range(heads):
                    wv = plsc.load_gather(wbufs[hd], [zi])
                    wcomb = jnp.where(lanes == hd, wv, wcomb)
                dv = plsc.load_gather(dstv, [zi])
                idx = dv * heads + jnp.where(lanes < heads, lanes, 0)
                row = jnp.right_shift(idx, 7)
                colv = jnp.bitwise_and(idx, 127)
                cur = plsc.load_gather(s_v, [row, colv])
                plsc.store_scatter(s_v, [row, colv], cur + wcomb,
                                   mask=lanes < heads)
                return carry2

            lax.fori_loop(0, EB, e_body, 0)
            return carry

        lax.fori_loop(0, n_chunks, chunk_body, 0)

        for bi, (base, sz) in enumerate(s_chunks):
            pltpu.sync_copy(s_v.at[pl.ds(base, sz)], s_sh.at[idxbufs[bi]],
                            add=True)

        plsc.subcore_barrier()

        @pl.when(s == 0)
        def _():
            pltpu.sync_copy(s_sh, ps_hbm.at[c])

    return pl.kernel(
        body,
        out_type=(jax.ShapeDtypeStruct((heads, e_pad), jnp.float32),
                  jax.ShapeDtypeStruct((NC, st_rows, 128), jnp.float32)),
        mesh=mesh,
        compiler_params=pltpu.CompilerParams(needs_layout_passes=False),
        scratch_types=[
            pltpu.VMEM_SHARED((st_rows, 128), jnp.float32),
            pltpu.VMEM((n_pad * heads,), jnp.float32),
            pltpu.VMEM((n_pad * heads,), jnp.float32),
            pltpu.VMEM((st_rows, 128), jnp.float32),
            pltpu.VMEM((EB,), jnp.int32),
            pltpu.VMEM((EB,), jnp.int32),
            [pltpu.VMEM((EB,), jnp.float32) for _ in range(heads)],
            [pltpu.VMEM((sz,), jnp.int32) for _, sz in s_chunks],
        ],
    ), st_rows


def _make_p_kernel(n_pad, e_pad, heads, EB=256):
    per_w = e_pad // (NC * NS)
    n_chunks = per_w // EB
    rows_per_tile = n_pad // NS
    cpg = 128 // heads // LN

    mesh = plsc.VectorSubcoreMesh(core_axis_name="c", subcore_axis_name="s")

    def body(src_hbm, dst_hbm, h_hbm, w_hbm, zeros_hbm,
             pay_hbm, acc_sh, srcv, dstv, grows, wrows, sem):
        c = lax.axis_index("c")
        s = lax.axis_index("s")
        wid = c * NS + s

        @pl.when(s == 0)
        def _():
            pltpu.sync_copy(zeros_hbm, acc_sh)

        plsc.subcore_barrier()

        def chunk_body(k, carry):
            ebase = wid * per_w + k * EB
            pltpu.sync_copy(src_hbm.at[pl.ds(ebase, EB)], srcv)
            pltpu.sync_copy(dst_hbm.at[pl.ds(ebase, EB)], dstv)
            for hd in range(heads):
                pltpu.sync_copy(w_hbm.at[hd, pl.ds(ebase, EB)], wrows[hd])
            pltpu.async_copy(h_hbm.at[srcv], grows, sem).wait()

            def e_body(e, carry2):
                zi = jnp.zeros((LN,), jnp.int32) + e
                for hd in range(heads):
                    wv = plsc.load_gather(wrows[hd], [zi])
                    for j in range(cpg):
                        col = (hd * cpg + j) * LN
                        grows[e, pl.ds(col, LN)] = (
                            grows[e, pl.ds(col, LN)] * wv)
                return carry2

            lax.fori_loop(0, EB, e_body, 0)

            pltpu.sync_copy(grows, acc_sh.at[dstv], add=True)
            return carry

        lax.fori_loop(0, n_chunks, chunk_body, 0)

        plsc.subcore_barrier()
        r0 = s * rows_per_tile
        pltpu.sync_copy(acc_sh.at[pl.ds(r0, rows_per_tile)],
                        pay_hbm.at[c, pl.ds(r0, rows_per_tile)])

    return pl.kernel(
        body,
        out_type=jax.ShapeDtypeStruct((NC, n_pad, 128), jnp.float32),
        mesh=mesh,
        compiler_params=pltpu.CompilerParams(needs_layout_passes=False),
        scratch_types=[
            pltpu.VMEM_SHARED((n_pad, 128), jnp.float32),
            pltpu.VMEM((EB,), jnp.int32),
            pltpu.VMEM((EB,), jnp.int32),
            pltpu.VMEM((EB, 128), jnp.float32),
            [pltpu.VMEM((EB,), jnp.float32) for _ in range(heads)],
            pltpu.SemaphoreType.DMA,
        ],
    )


def _tc_pre(x_ref, w_ref, as_ref, ad_ref, h_ref, asrc_ref, adst_ref):
    h = jnp.dot(x_ref[...], w_ref[...], preferred_element_type=jnp.float32)
    h_ref[...] = h
    asrc_ref[...] = jnp.dot(h, as_ref[...], preferred_element_type=jnp.float32)
    adst_ref[...] = jnp.dot(h, ad_ref[...], preferred_element_type=jnp.float32)


def _expand_heads(v, heads, width):
    rep = width // heads
    parts = [jnp.broadcast_to(v[:, hd:hd + 1], (v.shape[0], rep))
             for hd in range(heads)]
    return jnp.concatenate(parts, axis=1) if heads > 1 else parts[0]


def _tc_finalize(heads, n_valid, blk, with_next, part_ref, s0_ref, s1_ref,
                 h_ref, asrc_ref, adst_ref, b_ref, *rest):
    if with_next:
        w2_ref, as2_ref, ad2_ref, out_ref, asrc2_ref, adst2_ref = rest
    else:
        (out_ref,) = rest
    p = part_ref[0] + part_ref[1]
    al = asrc_ref[...] + adst_ref[...]
    wself = jnp.exp(jnp.where(al >= 0.0, al, al * 0.2))
    num = p + h_ref[...] * _expand_heads(wself, heads, 128)
    den = s0_ref[...] + s1_ref[...] + wself
    out = num / (_expand_heads(den, heads, 128) + 1e-16) + b_ref[...]
    rows = (lax.broadcasted_iota(jnp.int32, out.shape, 0)
            + pl.program_id(0) * blk)
    out = jnp.where(rows < n_valid, out, 0.0)
    if with_next:
        h2 = jnp.where(out > 0.0, out, jnp.exp(jnp.minimum(out, 0.0)) - 1.0)
        h2 = jnp.dot(h2, w2_ref[...], preferred_element_type=jnp.float32)
        h2 = jnp.where(rows < n_valid, h2, 0.0)
        out_ref[...] = h2
        asrc2_ref[...] = jnp.dot(h2, as2_ref[...],
                                 preferred_element_type=jnp.float32)
        adst2_ref[...] = jnp.dot(h2, ad2_ref[...],
                                 preferred_element_type=jnp.float32)
    else:
        out_ref[...] = out


def _att_mat(att):
    H = att.shape[0]
    return (att[:, :, None] * jnp.eye(H, dtype=att.dtype)[:, None, :]).reshape(
        att.shape[0] * att.shape[1], H)


def kernel(x, edge_index, W1, att_src1, att_dst1, b1, W2, att_src2, att_dst2,
           b2):
    N, D = x.shape
    E = edge_index.shape[1]
    H1 = att_src1.shape[0]
    H2 = att_src2.shape[0]
    n_pad = _ceil_to(N + 1, NS * 8)
    e_pad = _ceil_to(E, NC * NS * 256)

    x_pad = jnp.zeros((n_pad, D), x.dtype).at[:N].set(x)
    sent = jnp.full((e_pad - E,), N, jnp.int32)
    src = jnp.concatenate([edge_index[0], sent])
    dst = jnp.concatenate([edge_index[1], sent])
    zeros_acc = jnp.zeros((n_pad, 128), jnp.float32)

    A1s, A1d = _att_mat(att_src1), _att_mat(att_dst1)
    A2s, A2d = _att_mat(att_src2), _att_mat(att_dst2)

    F = W1.shape[1]
    BLK = 512 if n_pad % 512 == 0 else 128
    n_blk = n_pad // BLK

    def rblk(*trail):
        return pl.BlockSpec((BLK,) + trail, lambda i: (i,) + (0,) * len(trail))

    def full(shape):
        return pl.BlockSpec(shape, lambda i: (0,) * len(shape))

    h1, as1, ad1 = pl.pallas_call(
        _tc_pre,
        grid=(n_blk,),
        in_specs=[rblk(D), full((D, F)), full((F, H1)), full((F, H1))],
        out_specs=[rblk(F), rblk(H1), rblk(H1)],
        out_shape=[jax.ShapeDtypeStruct((n_pad, F), jnp.float32),
                   jax.ShapeDtypeStruct((n_pad, H1), jnp.float32),
                   jax.ShapeDtypeStruct((n_pad, H1), jnp.float32)],
    )(x_pad, W1, A1s, A1d)

    wk1, _ = _make_w_kernel(n_pad, e_pad, H1)
    w1, ps1 = wk1(src, dst, as1.reshape(-1), ad1.reshape(-1), zeros_acc)
    pk1 = _make_p_kernel(n_pad, e_pad, H1)
    pay1 = pk1(src, dst, h1, w1, zeros_acc)
    s1a = ps1[0].reshape(-1)[:n_pad * H1].reshape(n_pad, H1)
    s1b = ps1[1].reshape(-1)[:n_pad * H1].reshape(n_pad, H1)

    OUT = W2.shape[1]
    part_spec = pl.BlockSpec((NC, BLK, 128), lambda i: (0, i, 0))
    h2, as2, ad2 = pl.pallas_call(
        functools.partial(_tc_finalize, H1, N, BLK, True),
        grid=(n_blk,),
        in_specs=[part_spec, rblk(H1), rblk(H1), rblk(F), rblk(H1), rblk(H1),
                  full((1, F)), full((F, OUT)), full((OUT, H2)),
                  full((OUT, H2))],
        out_specs=[rblk(OUT), rblk(H2), rblk(H2)],
        out_shape=[jax.ShapeDtypeStruct((n_pad, OUT), jnp.float32),
                   jax.ShapeDtypeStruct((n_pad, H2), jnp.float32),
                   jax.ShapeDtypeStruct((n_pad, H2), jnp.float32)],
    )(pay1, s1a, s1b, h1, as1, ad1, b1.reshape(1, F), W2, A2s, A2d)

    wk2, _ = _make_w_kernel(n_pad, e_pad, H2)
    w2, ps2 = wk2(src, dst, as2.reshape(-1), ad2.reshape(-1), zeros_acc)
    pk2 = _make_p_kernel(n_pad, e_pad, H2)
    pay2 = pk2(src, dst, h2, w2, zeros_acc)
    s2a = ps2[0].reshape(-1)[:n_pad * H2].reshape(n_pad, H2)
    s2b = ps2[1].reshape(-1)[:n_pad * H2].reshape(n_pad, H2)

    out = pl.pallas_call(
        functools.partial(_tc_finalize, H2, N, BLK, False),
        grid=(n_blk,),
        in_specs=[part_spec, rblk(H2), rblk(H2), rblk(OUT), rblk(H2),
                  rblk(H2), full((1, OUT))],
        out_specs=rblk(OUT),
        out_shape=jax.ShapeDtypeStruct((n_pad, OUT), jnp.float32),
    )(pay2, s2a, s2b, h2, as2, ad2, b2.reshape(1, OUT))

    return out[:N]


# speedup vs baseline: 1.1836x; 1.1836x over previous
import functools

import jax
import jax.numpy as jnp
from jax import lax
from jax.experimental import pallas as pl
from jax.experimental.pallas import tpu as pltpu
from jax.experimental.pallas import tpu_sc as plsc

NC = 2
NS = 16
LN = 16
EB = 128


def _ceil_to(x, m):
    return (x + m - 1) // m * m


def _make_w_kernel(n_pad, e_pad, heads):
    per_w = e_pad // (NC * NS)
    n_chunks = per_w // EB
    st_rows = _ceil_to(_ceil_to(n_pad * heads, 128) // 128, NS)
    s_chunks = []
    b = 0
    while b < st_rows:
        sz = min(128, st_rows - b)
        s_chunks.append((b, sz))
        b += sz

    mesh = plsc.VectorSubcoreMesh(core_axis_name="c", subcore_axis_name="s")

    def body(src_hbm, dst_hbm, asrc_hbm, adst_hbm, zeros_hbm,
             w_hbm, ps_hbm, s_sh, asrc_v, adst_v, s_v, srcv, dstv,
             wbufs, idxbufs):
        c = lax.axis_index("c")
        s = lax.axis_index("s")
        wid = c * NS + s
        lanes = lax.broadcasted_iota(jnp.int32, (LN,), 0)
        zf = jnp.zeros((LN,), jnp.float32)

        pltpu.sync_copy(asrc_hbm, asrc_v)
        pltpu.sync_copy(adst_hbm, adst_v)

        @pl.when(s == 0)
        def _():
            pltpu.sync_copy(zeros_hbm.at[pl.ds(0, st_rows)], s_sh)

        def z_body(r, carry):
            for j in range(128 // LN):
                s_v[r, pl.ds(j * LN, LN)] = zf
            return carry

        lax.fori_loop(0, st_rows, z_body, 0)
        for bi, (base, sz) in enumerate(s_chunks):
            for g in range(sz // LN):
                idxbufs[bi][pl.ds(g * LN, LN)] = lanes + (base + g * LN)

        plsc.subcore_barrier()

        def chunk_body(k, carry):
            ebase = wid * per_w + k * EB
            pltpu.sync_copy(src_hbm.at[pl.ds(ebase, EB)], srcv)
            pltpu.sync_copy(dst_hbm.at[pl.ds(ebase, EB)], dstv)

            for g in range(EB // LN):
                s16 = srcv[pl.ds(g * LN, LN)]
                d16 = dstv[pl.ds(g * LN, LN)]
                for hd in range(heads):
                    a = plsc.load_gather(asrc_v, [s16 * heads + hd])
                    bb = plsc.load_gather(adst_v, [d16 * heads + hd])
                    al = a + bb
                    al = jnp.where(al >= 0.0, al, al * 0.2)
                    wbufs[hd][pl.ds(g * LN, LN)] = jnp.exp(al)
            for hd in range(heads):
                pltpu.sync_copy(wbufs[hd], w_hbm.at[hd, pl.ds(ebase, EB)])

            def e_body(e, carry2):
                zi = jnp.zeros((LN,), jnp.int32) + e
                wcomb = zf
                for hd in range(heads):
                    wv = plsc.load_gather(wbufs[hd], [zi])
                    wcomb = jnp.where(lanes == hd, wv, wcomb)
                dv = plsc.load_gather(dstv, [zi])
                idx = dv * heads + jnp.where(lanes < heads, lanes, 0)
                row = jnp.right_shift(idx, 7)
                colv = jnp.bitwise_and(idx, 127)
                cur = plsc.load_gather(s_v, [row, colv])
                plsc.store_scatter(s_v, [row, colv], cur + wcomb,
                                   mask=lanes < heads)
                return carry2

            lax.fori_loop(0, EB, e_body, 0)
            return carry

        lax.fori_loop(0, n_chunks, chunk_body, 0)

        for bi, (base, sz) in enumerate(s_chunks):
            pltpu.sync_copy(s_v.at[pl.ds(base, sz)], s_sh.at[idxbufs[bi]],
                            add=True)

        plsc.subcore_barrier()

        @pl.when(s == 0)
        def _():
            pltpu.sync_copy(s_sh, ps_hbm.at[c])

    return pl.kernel(
        body,
        out_type=(jax.ShapeDtypeStruct((heads, e_pad), jnp.float32),
                  jax.ShapeDtypeStruct((NC, st_rows, 128), jnp.float32)),
        mesh=mesh,
        compiler_params=pltpu.CompilerParams(needs_layout_passes=False),
        scratch_types=[
            pltpu.VMEM_SHARED((st_rows, 128), jnp.float32),
            pltpu.VMEM((n_pad * heads,), jnp.float32),
            pltpu.VMEM((n_pad * heads,), jnp.float32),
            pltpu.VMEM((st_rows, 128), jnp.float32),
            pltpu.VMEM((EB,), jnp.int32),
            pltpu.VMEM((EB,), jnp.int32),
            [pltpu.VMEM((EB,), jnp.float32) for _ in range(heads)],
            [pltpu.VMEM((sz,), jnp.int32) for _, sz in s_chunks],
        ],
    ), st_rows


def _make_p_kernel(n_pad, e_pad, heads, EB=128):
    per_w = e_pad // (NC * NS)
    n_chunks = per_w // EB
    rows_per_tile = n_pad // NS
    cpg = 128 // heads // LN

    mesh = plsc.VectorSubcoreMesh(core_axis_name="c", subcore_axis_name="s")

    def body(src_hbm, dst_hbm, h_hbm, w_hbm, zeros_hbm,
             pay_hbm, acc_sh, srcv, dstv, grows, wrows, sem):
        c = lax.axis_index("c")
        s = lax.axis_index("s")
        wid = c * NS + s

        @pl.when(s == 0)
        def _():
            pltpu.sync_copy(zeros_hbm, acc_sh)

        plsc.subcore_barrier()

        def chunk_body(k, carry):
            ebase = wid * per_w + k * EB
            pltpu.sync_copy(src_hbm.at[pl.ds(ebase, EB)], srcv)
            pltpu.sync_copy(dst_hbm.at[pl.ds(ebase, EB)], dstv)
            for hd in range(heads):
                pltpu.sync_copy(w_hbm.at[hd, pl.ds(ebase, EB)], wrows[hd])
            pltpu.async_copy(h_hbm.at[srcv], grows, sem).wait()

            def e_body(e, carry2):
                zi = jnp.zeros((LN,), jnp.int32) + e
                for hd in range(heads):
                    wv = plsc.load_gather(wrows[hd], [zi])
                    for j in range(cpg):
                        col = (hd * cpg + j) * LN
                        grows[e, pl.ds(col, LN)] = (
                            grows[e, pl.ds(col, LN)] * wv)
                return carry2

            lax.fori_loop(0, EB, e_body, 0)

            pltpu.sync_copy(grows, acc_sh.at[dstv], add=True)
            return carry

        lax.fori_loop(0, n_chunks, chunk_body, 0)

        plsc.subcore_barrier()
        r0 = s * rows_per_tile
        pltpu.sync_copy(acc_sh.at[pl.ds(r0, rows_per_tile)],
                        pay_hbm.at[c, pl.ds(r0, rows_per_tile)])

    return pl.kernel(
        body,
        out_type=jax.ShapeDtypeStruct((NC, n_pad, 128), jnp.float32),
        mesh=mesh,
        compiler_params=pltpu.CompilerParams(needs_layout_passes=False),
        scratch_types=[
            pltpu.VMEM_SHARED((n_pad, 128), jnp.float32),
            pltpu.VMEM((EB,), jnp.int32),
            pltpu.VMEM((EB,), jnp.int32),
            pltpu.VMEM((EB, 128), jnp.float32),
            [pltpu.VMEM((EB,), jnp.float32) for _ in range(heads)],
            pltpu.SemaphoreType.DMA,
        ],
    )


def _tc_pre(x_ref, w_ref, as_ref, ad_ref, h_ref, asrc_ref, adst_ref):
    h = jnp.dot(x_ref[...], w_ref[...], preferred_element_type=jnp.float32)
    h_ref[...] = h
    asrc_ref[...] = jnp.dot(h, as_ref[...], preferred_element_type=jnp.float32)
    adst_ref[...] = jnp.dot(h, ad_ref[...], preferred_element_type=jnp.float32)


def _expand_heads(v, heads, width):
    rep = width // heads
    parts = [jnp.broadcast_to(v[:, hd:hd + 1], (v.shape[0], rep))
             for hd in range(heads)]
    return jnp.concatenate(parts, axis=1) if heads > 1 else parts[0]


def _tc_finalize(heads, n_valid, blk, with_next, part_ref, s0_ref, s1_ref,
                 h_ref, asrc_ref, adst_ref, b_ref, *rest):
    if with_next:
        w2_ref, as2_ref, ad2_ref, out_ref, asrc2_ref, adst2_ref = rest
    else:
        (out_ref,) = rest
    p = part_ref[0] + part_ref[1]
    al = asrc_ref[...] + adst_ref[...]
    wself = jnp.exp(jnp.where(al >= 0.0, al, al * 0.2))
    num = p + h_ref[...] * _expand_heads(wself, heads, 128)
    den = s0_ref[...] + s1_ref[...] + wself
    out = num / (_expand_heads(den, heads, 128) + 1e-16) + b_ref[...]
    rows = (lax.broadcasted_iota(jnp.int32, out.shape, 0)
            + pl.program_id(0) * blk)
    out = jnp.where(rows < n_valid, out, 0.0)
    if with_next:
        h2 = jnp.where(out > 0.0, out, jnp.exp(jnp.minimum(out, 0.0)) - 1.0)
        h2 = jnp.dot(h2, w2_ref[...], preferred_element_type=jnp.float32)
        h2 = jnp.where(rows < n_valid, h2, 0.0)
        out_ref[...] = h2
        asrc2_ref[...] = jnp.dot(h2, as2_ref[...],
                                 preferred_element_type=jnp.float32)
        adst2_ref[...] = jnp.dot(h2, ad2_ref[...],
                                 preferred_element_type=jnp.float32)
    else:
        out_ref[...] = out


def _att_mat(att):
    H = att.shape[0]
    return (att[:, :, None] * jnp.eye(H, dtype=att.dtype)[:, None, :]).reshape(
        att.shape[0] * att.shape[1], H)


def kernel(x, edge_index, W1, att_src1, att_dst1, b1, W2, att_src2, att_dst2,
           b2):
    N, D = x.shape
    E = edge_index.shape[1]
    H1 = att_src1.shape[0]
    H2 = att_src2.shape[0]
    n_pad = _ceil_to(N + 1, NS * 8)
    e_pad = _ceil_to(E, NC * NS * EB)

    x_pad = jnp.zeros((n_pad, D), x.dtype).at[:N].set(x)
    sent = jnp.full((e_pad - E,), N, jnp.int32)
    src = jnp.concatenate([edge_index[0], sent])
    dst = jnp.concatenate([edge_index[1], sent])
    zeros_acc = jnp.zeros((n_pad, 128), jnp.float32)

    A1s, A1d = _att_mat(att_src1), _att_mat(att_dst1)
    A2s, A2d = _att_mat(att_src2), _att_mat(att_dst2)

    F = W1.shape[1]
    BLK = 512 if n_pad % 512 == 0 else 128
    n_blk = n_pad // BLK

    def rblk(*trail):
        return pl.BlockSpec((BLK,) + trail, lambda i: (i,) + (0,) * len(trail))

    def full(shape):
        return pl.BlockSpec(shape, lambda i: (0,) * len(shape))

    h1, as1, ad1 = pl.pallas_call(
        _tc_pre,
        grid=(n_blk,),
        in_specs=[rblk(D), full((D, F)), full((F, H1)), full((F, H1))],
        out_specs=[rblk(F), rblk(H1), rblk(H1)],
        out_shape=[jax.ShapeDtypeStruct((n_pad, F), jnp.float32),
                   jax.ShapeDtypeStruct((n_pad, H1), jnp.float32),
                   jax.ShapeDtypeStruct((n_pad, H1), jnp.float32)],
    )(x_pad, W1, A1s, A1d)

    wk1, _ = _make_w_kernel(n_pad, e_pad, H1)
    w1, ps1 = wk1(src, dst, as1.reshape(-1), ad1.reshape(-1), zeros_acc)
    pk1 = _make_p_kernel(n_pad, e_pad, H1)
    pay1 = pk1(src, dst, h1, w1, zeros_acc)
    s1a = ps1[0].reshape(-1)[:n_pad * H1].reshape(n_pad, H1)
    s1b = ps1[1].reshape(-1)[:n_pad * H1].reshape(n_pad, H1)

    OUT = W2.shape[1]
    part_spec = pl.BlockSpec((NC, BLK, 128), lambda i: (0, i, 0))
    h2, as2, ad2 = pl.pallas_call(
        functools.partial(_tc_finalize, H1, N, BLK, True),
        grid=(n_blk,),
        in_specs=[part_spec, rblk(H1), rblk(H1), rblk(F), rblk(H1), rblk(H1),
                  full((1, F)), full((F, OUT)), full((OUT, H2)),
                  full((OUT, H2))],
        out_specs=[rblk(OUT), rblk(H2), rblk(H2)],
        out_shape=[jax.ShapeDtypeStruct((n_pad, OUT), jnp.float32),
                   jax.ShapeDtypeStruct((n_pad, H2), jnp.float32),
                   jax.ShapeDtypeStruct((n_pad, H2), jnp.float32)],
    )(pay1, s1a, s1b, h1, as1, ad1, b1.reshape(1, F), W2, A2s, A2d)

    wk2, _ = _make_w_kernel(n_pad, e_pad, H2)
    w2, ps2 = wk2(src, dst, as2.reshape(-1), ad2.reshape(-1), zeros_acc)
    pk2 = _make_p_kernel(n_pad, e_pad, H2)
    pay2 = pk2(src, dst, h2, w2, zeros_acc)
    s2a = ps2[0].reshape(-1)[:n_pad * H2].reshape(n_pad, H2)
    s2b = ps2[1].reshape(-1)[:n_pad * H2].reshape(n_pad, H2)

    out = pl.pallas_call(
        functools.partial(_tc_finalize, H2, N, BLK, False),
        grid=(n_blk,),
        in_specs=[part_spec, rblk(H2), rblk(H2), rblk(OUT), rblk(H2),
                  rblk(H2), full((1, OUT))],
        out_specs=rblk(OUT),
        out_shape=jax.ShapeDtypeStruct((n_pad, OUT), jnp.float32),
    )(pay2, s2a, s2b, h2, as2, ad2, b2.reshape(1, OUT))

    return out[:N]
